# Initial kernel scaffold; baseline (speedup 1.0000x reference)
import jax
import jax.numpy as jnp
from jax.experimental import pallas as pl


def kernel(vecs, loss_mask, c_sum, c_count, n_device, n_block_per_update):
    raise NotImplementedError("write your pallas kernel here")


# speedup vs baseline: 269.5867x; 269.5867x over previous
import functools

import jax
import jax.numpy as jnp
from jax import lax
from jax.experimental import pallas as pl
from jax.experimental.pallas import tpu as pltpu
from jax.experimental.pallas import tpu_sc as plsc

_B = 4
_H = 1
_L = 4096
_DK = 128
_S = 512
_N = _B * _H * _L
_GAMMA = 0.99

_BLK = 1024
_NB = _N // _BLK

_NC = 2
_NS = 16
_NW = _NC * _NS
_TPW = _N // _NW
_CH = 128
_NCH = _TPW // _CH
_SPAD = 640
_RPT_Z = _SPAD // _NS
_RPT_O = _S // _NS


def _dist_body(v_ref, lm_ref, csum_ref, ccnt_ref,
               z_ref, e_ref, c_ref, commit_ref, cnt_ref):
    i = pl.program_id(0)
    cc = jnp.clip(ccnt_ref[0, :], 0.01, None)
    c = csum_ref[...] / cc[:, None]
    v = v_ref[...]
    scores = lax.dot_general(
        v, c, dimension_numbers=(((1,), (1,)), ((), ())),
        preferred_element_type=jnp.float32)
    vsq = jnp.sum(v * v, axis=1, keepdims=True)
    cbsq = jnp.sum(c * c, axis=1)
    d = vsq - 2.0 * scores + cbsq[None, :]
    m = jnp.min(d, axis=1, keepdims=True)
    iota = lax.broadcasted_iota(jnp.int32, (_BLK, _S), 1)
    z = jnp.min(jnp.where(d == m, iota, _S), axis=1)
    e = jnp.maximum(m[:, 0], 0.0)
    z_ref[0, 0, :] = z
    e_ref[0, 0, :] = e

    lm = lm_ref[0, 0, :].astype(jnp.float32)
    part_commit = jnp.sum(e * lm)
    onehot = jnp.where(
        (iota == z[:, None]) & (lm[:, None] != 0.0), 1.0, 0.0)
    part_cnt = jnp.sum(onehot, axis=0)

    @pl.when(i == 0)
    def _():
        c_ref[...] = c
        commit_ref[0, 0] = 0.0
        cnt_ref[...] = jnp.zeros((1, _S), jnp.float32)

    commit_ref[0, 0] += part_commit
    cnt_ref[0, :] += part_cnt


_dist_call = pl.pallas_call(
    _dist_body,
    grid=(_NB,),
    in_specs=[
        pl.BlockSpec((_BLK, _DK), lambda i: (i, 0)),
        pl.BlockSpec((1, 1, _BLK), lambda i: (i, 0, 0)),
        pl.BlockSpec((_S, _DK), lambda i: (0, 0)),
        pl.BlockSpec((1, _S), lambda i: (0, 0)),
    ],
    out_specs=[
        pl.BlockSpec((1, 1, _BLK), lambda i: (i, 0, 0)),
        pl.BlockSpec((1, 1, _BLK), lambda i: (i, 0, 0)),
        pl.BlockSpec((_S, _DK), lambda i: (0, 0)),
        pl.BlockSpec(memory_space=pltpu.SMEM),
        pl.BlockSpec((1, _S), lambda i: (0, 0)),
    ],
    out_shape=[
        jax.ShapeDtypeStruct((_NB, 1, _BLK), jnp.int32),
        jax.ShapeDtypeStruct((_NB, 1, _BLK), jnp.float32),
        jax.ShapeDtypeStruct((_S, _DK), jnp.float32),
        jax.ShapeDtypeStruct((1, 1), jnp.float32),
        jax.ShapeDtypeStruct((1, _S), jnp.float32),
    ],
    compiler_params=pltpu.CompilerParams(
        dimension_semantics=("arbitrary",)),
)


def _sc_body(c_hbm, z_hbm, lm_hbm, vecs_hbm, zeros_hbm,
             vh_hbm, csum_hbm,
             idxg, idxs, lmbuf, rows, vbuf, acc, sem):
    cid = lax.axis_index("c")
    sid = lax.axis_index("s")
    wid = sid * _NC + cid

    pltpu.sync_copy(zeros_hbm.at[pl.ds(sid * _RPT_Z, _RPT_Z)],
                    acc.at[pl.ds(sid * _RPT_Z, _RPT_Z)])
    plsc.subcore_barrier()

    def chunk(ch, carry):
        base = wid * _TPW + ch * _CH
        pltpu.sync_copy(z_hbm.at[pl.ds(base, _CH)], idxg)
        pltpu.async_copy(c_hbm.at[idxg], rows, sem).wait()
        pltpu.sync_copy(rows, vh_hbm.at[pl.ds(base, _CH)])
        pltpu.sync_copy(lm_hbm.at[pl.ds(base, _CH)], lmbuf)

        def mk(j, c2):
            zv = idxg[pl.ds(j * 16, 16)]
            lv = lmbuf[pl.ds(j * 16, 16)]
            idxs[pl.ds(j * 16, 16)] = jnp.where(lv == 0, _S, zv)
            return c2
        lax.fori_loop(0, _CH // 16, mk, 0)

        pltpu.sync_copy(vecs_hbm.at[pl.ds(base, _CH)], vbuf)
        pltpu.sync_copy(vbuf, acc.at[idxs], add=True)
        return carry

    lax.fori_loop(0, _NCH, chunk, 0)
    plsc.subcore_barrier()
    pltpu.sync_copy(acc.at[pl.ds(sid * _RPT_O, _RPT_O)],
                    csum_hbm.at[cid, pl.ds(sid * _RPT_O, _RPT_O)])


@functools.lru_cache(maxsize=1)
def _get_sc_call():
    return functools.partial(
        pl.kernel,
        mesh=plsc.VectorSubcoreMesh(core_axis_name="c", subcore_axis_name="s"),
        out_type=[
            jax.ShapeDtypeStruct((_N, _DK), jnp.float32),
            jax.ShapeDtypeStruct((_NC, _S, _DK), jnp.float32),
        ],
        scratch_types=[
            pltpu.VMEM((_CH,), jnp.int32),
            pltpu.VMEM((_CH,), jnp.int32),
            pltpu.VMEM((_CH,), jnp.int32),
            pltpu.VMEM((_CH, _DK), jnp.float32),
            pltpu.VMEM((_CH, _DK), jnp.float32),
            pltpu.VMEM_SHARED((_SPAD, _DK), jnp.float32),
            pltpu.SemaphoreType.DMA,
        ],
    )(_sc_body)


def _loss_body(csp_ref, csum_ref, ccnt_ref, cnt_ref, commit_ref, scale_ref,
               lcm_ref, lcb_ref):
    sc = scale_ref[0, 0]
    c_sum_hat = sc * (csp_ref[0] + csp_ref[1])
    c_count_hat = sc * cnt_ref[0, :]
    cs = csum_ref[...]
    cn = ccnt_ref[0, :]
    c_sum_tgt = (1.0 - _GAMMA) * c_sum_hat + _GAMMA * cs
    c_count_tgt = (1.0 - _GAMMA) * c_count_hat + _GAMMA * cn
    lcb = jnp.sum((cs - c_sum_tgt) * cs) + jnp.sum((cn - c_count_tgt) * cn)
    lcb_ref[0, 0] = lcb
    lcm_ref[0, 0] = commit_ref[0, 0] * (1.0 / float(_N))


_loss_call = pl.pallas_call(
    _loss_body,
    in_specs=[
        pl.BlockSpec(memory_space=pltpu.VMEM),
        pl.BlockSpec(memory_space=pltpu.VMEM),
        pl.BlockSpec(memory_space=pltpu.VMEM),
        pl.BlockSpec(memory_space=pltpu.VMEM),
        pl.BlockSpec(memory_space=pltpu.SMEM),
        pl.BlockSpec(memory_space=pltpu.SMEM),
    ],
    out_specs=[
        pl.BlockSpec(memory_space=pltpu.SMEM),
        pl.BlockSpec(memory_space=pltpu.SMEM),
    ],
    out_shape=[
        jax.ShapeDtypeStruct((1, 1), jnp.float32),
        jax.ShapeDtypeStruct((1, 1), jnp.float32),
    ],
)


def kernel(vecs, loss_mask, c_sum, c_count, n_device, n_block_per_update):
    v2 = vecs.reshape(_N, _DK)
    lm3 = loss_mask.reshape(_NB, 1, _BLK)
    cs2 = c_sum.reshape(_S, _DK)
    cc2 = c_count.reshape(1, _S)

    z3, e3, c, commit, cnt = _dist_call(v2, lm3, cs2, cc2)

    zeros = jnp.zeros((_SPAD, _DK), jnp.float32)
    vh, csp = _get_sc_call()(c, z3.reshape(_N), loss_mask.reshape(_N), v2, zeros)

    scale = (jnp.asarray(n_device, jnp.float32)
             * jnp.asarray(n_block_per_update, jnp.float32)).reshape(1, 1)
    lcm, lcb = _loss_call(csp, cs2, cc2, cnt, commit, scale)

    vecs_hat = vh.reshape(_B, _H, _L, _DK)
    z = z3.reshape(_B, _H, _L)
    errs2 = e3.reshape(_B, _H, _L)
    return vecs_hat, z, lcm[0, 0], lcb[0, 0], errs2


# speedup vs baseline: 289.2847x; 1.0731x over previous
import functools

import jax
import jax.numpy as jnp
from jax import lax
from jax.experimental import pallas as pl
from jax.experimental.pallas import tpu as pltpu
from jax.experimental.pallas import tpu_sc as plsc

_B = 4
_H = 1
_L = 4096
_DK = 128
_S = 512
_N = _B * _H * _L
_GAMMA = 0.99

_BLK = 1024
_NB = _N // _BLK

_NC = 2
_NS = 16
_NW = _NC * _NS
_TPW = _N // _NW
_CH = 128
_NCH = _TPW // _CH


def _dist_body(v_ref, lm_ref, csum_ref, ccnt_ref, scale_ref,
               z_ref, e_ref, c_ref, lcm_ref, lcb_ref):
    i = pl.program_id(0)
    cn = ccnt_ref[0, :]
    cc = jnp.clip(cn, 0.01, None)
    c = csum_ref[...] / cc[:, None]
    v = v_ref[...]
    scores = lax.dot_general(
        v, c, dimension_numbers=(((1,), (1,)), ((), ())),
        preferred_element_type=jnp.float32)
    vsq = jnp.sum(v * v, axis=1, keepdims=True)
    cbsq = jnp.sum(c * c, axis=1)
    d = vsq - 2.0 * scores + cbsq[None, :]
    m = jnp.min(d, axis=1, keepdims=True)
    iota = lax.broadcasted_iota(jnp.int32, (_BLK, _S), 1)
    z = jnp.min(jnp.where(d == m, iota, _S), axis=1)
    e = jnp.maximum(m[:, 0], 0.0)
    z_ref[0, 0, :] = z
    e_ref[0, 0, :] = e

    lm = lm_ref[0, 0, :].astype(jnp.float32)
    oh = jnp.where(iota == z[:, None], 1.0, 0.0)
    u = jnp.stack([lm, lm * (vsq[:, 0] - m[:, 0]) * 0.5], axis=0)
    c2 = lax.dot_general(
        u, oh, dimension_numbers=(((1,), (0,)), ((), ())),
        preferred_element_type=jnp.float32)
    g = 0.5 * (cc * cbsq) + cn
    part_cb = jnp.sum(c2[0, :] * g + c2[1, :] * cc)
    part_commit = jnp.sum(e * lm)

    @pl.when(i == 0)
    def _():
        c_ref[...] = c
        lcm_ref[0, 0] = 0.0
        lcb_ref[0, 0] = 0.0

    lcm_ref[0, 0] += part_commit
    lcb_ref[0, 0] += part_cb

    @pl.when(i == _NB - 1)
    def _():
        cs = csum_ref[...]
        s2 = jnp.sum(cs * cs) + jnp.sum(cn * cn)
        sc = scale_ref[0, 0]
        lcb_ref[0, 0] = (1.0 - _GAMMA) * (s2 - sc * lcb_ref[0, 0])
        lcm_ref[0, 0] = lcm_ref[0, 0] * (1.0 / float(_N))


_dist_call = pl.pallas_call(
    _dist_body,
    grid=(_NB,),
    in_specs=[
        pl.BlockSpec((_BLK, _DK), lambda i: (i, 0)),
        pl.BlockSpec((1, 1, _BLK), lambda i: (i, 0, 0)),
        pl.BlockSpec((_S, _DK), lambda i: (0, 0)),
        pl.BlockSpec((1, _S), lambda i: (0, 0)),
        pl.BlockSpec(memory_space=pltpu.SMEM),
    ],
    out_specs=[
        pl.BlockSpec((1, 1, _BLK), lambda i: (i, 0, 0)),
        pl.BlockSpec((1, 1, _BLK), lambda i: (i, 0, 0)),
        pl.BlockSpec((_S, _DK), lambda i: (0, 0)),
        pl.BlockSpec(memory_space=pltpu.SMEM),
        pl.BlockSpec(memory_space=pltpu.SMEM),
    ],
    out_shape=[
        jax.ShapeDtypeStruct((_NB, 1, _BLK), jnp.int32),
        jax.ShapeDtypeStruct((_NB, 1, _BLK), jnp.float32),
        jax.ShapeDtypeStruct((_S, _DK), jnp.float32),
        jax.ShapeDtypeStruct((1, 1), jnp.float32),
        jax.ShapeDtypeStruct((1, 1), jnp.float32),
    ],
    compiler_params=pltpu.CompilerParams(
        dimension_semantics=("arbitrary",)),
)


def _sc_body(c_hbm, z_hbm, vh_hbm, idxall, rows, sem):
    cid = lax.axis_index("c")
    sid = lax.axis_index("s")
    wid = sid * _NC + cid
    base = wid * _TPW
    pltpu.sync_copy(z_hbm.at[pl.ds(base, _TPW)], idxall)
    copies = []
    for k in range(_NCH):
        copies.append(pltpu.async_copy(
            c_hbm.at[idxall.at[pl.ds(k * _CH, _CH)]],
            rows.at[pl.ds(k * _CH, _CH)], sem))
    for cp in copies:
        cp.wait()
    pltpu.sync_copy(rows, vh_hbm.at[pl.ds(base, _TPW)])


@functools.lru_cache(maxsize=1)
def _get_sc_call():
    return functools.partial(
        pl.kernel,
        mesh=plsc.VectorSubcoreMesh(core_axis_name="c", subcore_axis_name="s"),
        out_type=jax.ShapeDtypeStruct((_N, _DK), jnp.float32),
        scratch_types=[
            pltpu.VMEM((_TPW,), jnp.int32),
            pltpu.VMEM((_TPW, _DK), jnp.float32),
            pltpu.SemaphoreType.DMA,
        ],
    )(_sc_body)


def kernel(vecs, loss_mask, c_sum, c_count, n_device, n_block_per_update):
    v2 = vecs.reshape(_N, _DK)
    lm3 = loss_mask.reshape(_NB, 1, _BLK)
    cs2 = c_sum.reshape(_S, _DK)
    cc2 = c_count.reshape(1, _S)
    scale = (jnp.asarray(n_device, jnp.float32)
             * jnp.asarray(n_block_per_update, jnp.float32)).reshape(1, 1)

    z3, e3, c, lcm, lcb = _dist_call(v2, lm3, cs2, cc2, scale)
    vh = _get_sc_call()(c, z3.reshape(_N))

    vecs_hat = vh.reshape(_B, _H, _L, _DK)
    z = z3.reshape(_B, _H, _L)
    errs2 = e3.reshape(_B, _H, _L)
    return vecs_hat, z, lcm[0, 0], lcb[0, 0], errs2
